# baseline (speedup 1.0000x reference)
import jax
import jax.numpy as jnp
from jax import lax
from jax.experimental import pallas as pl
from jax.experimental.pallas import tpu as pltpu
from jax.experimental.pallas import tpu_sc as plsc

Fh = 64
Fv = 64
Fe = 16
B_EVENTS = 4096
N_SLOTS = 512
N_PAIRS = N_SLOTS // 2
PAIR_ROWS = B_EVENTS * N_PAIRS

SC_CORES = 2
SC_SUBCORES = 16
NW = SC_CORES * SC_SUBCORES
ROWS_PER_W = B_EVENTS // NW


def _sc_gather_body(hrows_hbm, esrc_hbm, edst_hbm, hsrc_out, hdst_out,
                    ev_src, ev_dst, idx_src, idx_dst, rows_src, rows_dst,
                    sem_a, sem_b):
    wid = lax.axis_index("s") * SC_CORES + lax.axis_index("c")
    base = wid * ROWS_PER_W
    pltpu.sync_copy(esrc_hbm.at[pl.ds(base, ROWS_PER_W)], ev_src)
    pltpu.sync_copy(edst_hbm.at[pl.ds(base, ROWS_PER_W)], ev_dst)
    lane = lax.iota(jnp.int32, 16)
    for k in range(ROWS_PER_W // 16):
        row0 = (base + k * 16) * N_SLOTS
        idx_src[pl.ds(k * 16, 16)] = lax.shift_right_logical(
            ev_src[pl.ds(k * 16, 16)] + lane * N_SLOTS + row0, 1)
        idx_dst[pl.ds(k * 16, 16)] = lax.shift_right_logical(
            ev_dst[pl.ds(k * 16, 16)] + lane * N_SLOTS + row0, 1)
    cp_a = pltpu.async_copy(hrows_hbm.at[idx_src], rows_src, sem_a)
    cp_b = pltpu.async_copy(hrows_hbm.at[idx_dst], rows_dst, sem_b)
    cp_a.wait()
    cp_b.wait()
    pltpu.sync_copy(rows_src, hsrc_out.at[pl.ds(base, ROWS_PER_W)])
    pltpu.sync_copy(rows_dst, hdst_out.at[pl.ds(base, ROWS_PER_W)])


def _sc_gather(hrows, esrc, edst):
    mesh = plsc.VectorSubcoreMesh(core_axis_name="c", subcore_axis_name="s")
    f32 = jnp.float32
    return pl.kernel(
        _sc_gather_body,
        out_type=[jax.ShapeDtypeStruct((B_EVENTS, 2 * Fh), f32),
                  jax.ShapeDtypeStruct((B_EVENTS, 2 * Fh), f32)],
        mesh=mesh,
        scratch_types=[
            pltpu.VMEM((ROWS_PER_W,), jnp.int32),
            pltpu.VMEM((ROWS_PER_W,), jnp.int32),
            pltpu.VMEM((ROWS_PER_W,), jnp.int32),
            pltpu.VMEM((ROWS_PER_W,), jnp.int32),
            pltpu.VMEM((ROWS_PER_W, 2 * Fh), f32),
            pltpu.VMEM((ROWS_PER_W, 2 * Fh), f32),
            pltpu.SemaphoreType.DMA,
            pltpu.SemaphoreType.DMA,
        ],
    )(hrows, esrc, edst)


def _gru_gates(gi, gh, h):
    i_r, i_z, i_n = gi[:, :Fh], gi[:, Fh:2 * Fh], gi[:, 2 * Fh:]
    h_r, h_z, h_n = gh[:, :Fh], gh[:, Fh:2 * Fh], gh[:, 2 * Fh:]
    r = jax.nn.sigmoid(i_r + h_r)
    z = jax.nn.sigmoid(i_z + h_z)
    n = jnp.tanh(i_n + r * h_n)
    return (1.0 - z) * n + z * h


def _gru_body(pair_src_ref, pair_dst_ref, psrc_ref, pdst_ref, same_ref,
              xsh_ref,
              w1o_ref, w2o_ref, who_ref, bio_ref, bho_ref,
              w1i_ref, w2i_ref, whi_ref, bii_ref, bhi_ref,
              hsrc_new_ref, hdst_new_ref, pso_ref, pdo_ref):
    ps = psrc_ref[...] == 1
    pd = pdst_ref[...] == 1
    same = same_ref[...] == 1
    src_lo, src_hi = pair_src_ref[:, :Fh], pair_src_ref[:, Fh:]
    dst_lo, dst_hi = pair_dst_ref[:, :Fh], pair_dst_ref[:, Fh:]
    hs = jnp.where(ps, src_hi, src_lo)
    hd = jnp.where(pd, dst_hi, dst_lo)
    xsh = xsh_ref[...]
    f32 = jnp.float32
    gi = (jnp.dot(hd, w1o_ref[...], preferred_element_type=f32)
          + jnp.dot(xsh, w2o_ref[...], preferred_element_type=f32)
          + bio_ref[...])
    gh = jnp.dot(hs, who_ref[...], preferred_element_type=f32) + bho_ref[...]
    hs_new = _gru_gates(gi, gh, hs)
    hsrc_new_ref[...] = hs_new
    gi = (jnp.dot(hs, w1i_ref[...], preferred_element_type=f32)
          + jnp.dot(xsh, w2i_ref[...], preferred_element_type=f32)
          + bii_ref[...])
    gh = jnp.dot(hd, whi_ref[...], preferred_element_type=f32) + bhi_ref[...]
    hd_new = _gru_gates(gi, gh, hd)
    hdst_new_ref[...] = hd_new
    pso_ref[...] = jnp.concatenate(
        [jnp.where(ps, src_lo, hs_new), jnp.where(ps, hs_new, src_hi)],
        axis=1)
    out_lo = jnp.where(same & (~ps) & pd, hs_new, dst_lo)
    out_hi = jnp.where(same & ps & (~pd), hs_new, dst_hi)
    out_lo = jnp.where(~pd, hd_new, out_lo)
    out_hi = jnp.where(pd, hd_new, out_hi)
    pdo_ref[...] = jnp.concatenate([out_lo, out_hi], axis=1)


def _tc_gru(pair_src, pair_dst, psrc, pdst, same, xshared,
            w1o, w2o, who, bio, bho, w1i, w2i, whi, bii, bhi):
    bb = 1024
    grid = (B_EVENTS // bb,)
    row_blk = lambda w: pl.BlockSpec((bb, w), lambda b: (b, 0))
    full = lambda a: pl.BlockSpec(a.shape, lambda b: (0,) * a.ndim)
    f32 = jnp.float32
    return pl.pallas_call(
        _gru_body,
        grid=grid,
        in_specs=[row_blk(2 * Fh), row_blk(2 * Fh),
                  row_blk(1), row_blk(1), row_blk(1),
                  row_blk(2 * Fv + Fe),
                  full(w1o), full(w2o), full(who), full(bio), full(bho),
                  full(w1i), full(w2i), full(whi), full(bii), full(bhi)],
        out_specs=[row_blk(Fh), row_blk(Fh),
                   row_blk(2 * Fh), row_blk(2 * Fh)],
        out_shape=[jax.ShapeDtypeStruct((B_EVENTS, Fh), f32),
                   jax.ShapeDtypeStruct((B_EVENTS, Fh), f32),
                   jax.ShapeDtypeStruct((B_EVENTS, 2 * Fh), f32),
                   jax.ShapeDtypeStruct((B_EVENTS, 2 * Fh), f32)],
    )(pair_src, pair_dst, psrc, pdst, same, xshared,
      w1o, w2o, who, bio, bho, w1i, w2i, whi, bii, bhi)


_SCAT_BB = 8


def _scatter_body(ps_ref, pd_ref, h_ref, pso_ref, pdo_ref, out_ref):
    out_ref[...] = h_ref[...]
    for i in range(_SCAT_BB):
        s = ps_ref[0, 0, i]
        t = pd_ref[0, 0, i]
        out_ref[i, pl.ds(s, 1), :] = pso_ref[pl.ds(i, 1), :]
        out_ref[i, pl.ds(t, 1), :] = pdo_ref[pl.ds(i, 1), :]


def _tc_scatter(hp, pair_src_out, pair_dst_out, psrc_pair, pdst_pair):
    nb = B_EVENTS // _SCAT_BB
    e3_src = psrc_pair.reshape(nb, 1, _SCAT_BB)
    e3_dst = pdst_pair.reshape(nb, 1, _SCAT_BB)
    smem_blk = pl.BlockSpec((1, 1, _SCAT_BB), lambda b: (b, 0, 0),
                            memory_space=pltpu.SMEM)
    return pl.pallas_call(
        _scatter_body,
        grid=(nb,),
        in_specs=[smem_blk, smem_blk,
                  pl.BlockSpec((_SCAT_BB, N_PAIRS, 2 * Fh),
                               lambda b: (b, 0, 0)),
                  pl.BlockSpec((_SCAT_BB, 2 * Fh), lambda b: (b, 0)),
                  pl.BlockSpec((_SCAT_BB, 2 * Fh), lambda b: (b, 0))],
        out_specs=pl.BlockSpec((_SCAT_BB, N_PAIRS, 2 * Fh),
                               lambda b: (b, 0, 0)),
        out_shape=jax.ShapeDtypeStruct((B_EVENTS, N_PAIRS, 2 * Fh),
                                       jnp.float32),
        compiler_params=pltpu.CompilerParams(
            dimension_semantics=("arbitrary",)),
    )(e3_src, e3_dst, hp, pair_src_out, pair_dst_out)


def kernel(E, Xe, Xv, H, Wih_out, Whh_out, bih_out, bhh_out,
           Wih_in, Whh_in, bih_in, bhh_in):
    esrc = E[:, 0]
    edst = E[:, 1]
    hrows = H.reshape(PAIR_ROWS, 2 * Fh)

    pair_src, pair_dst = _sc_gather(hrows, esrc, edst)

    xshared = jnp.concatenate([Xv[:, 0, :], Xv[:, 1, :], Xe], axis=1)
    psrc = (esrc & 1)[:, None]
    pdst = (edst & 1)[:, None]
    same = ((esrc >> 1) == (edst >> 1)).astype(jnp.int32)[:, None]
    w1o = Wih_out[:, :Fh].T
    w2o = Wih_out[:, Fh:].T
    w1i = Wih_in[:, :Fh].T
    w2i = Wih_in[:, Fh:].T
    hsrc_new, hdst_new, pair_src_out, pair_dst_out = _tc_gru(
        pair_src, pair_dst, psrc, pdst, same, xshared,
        w1o, w2o, Whh_out.T, bih_out[None, :], bhh_out[None, :],
        w1i, w2i, Whh_in.T, bih_in[None, :], bhh_in[None, :])

    hp = H.reshape(B_EVENTS, N_PAIRS, 2 * Fh)
    h_out = _tc_scatter(hp, pair_src_out, pair_dst_out,
                        esrc >> 1, edst >> 1)
    return (hsrc_new, hdst_new, h_out.reshape(B_EVENTS, N_SLOTS, Fh))


# speedup vs baseline: 4.7894x; 4.7894x over previous
import jax
import jax.numpy as jnp
from jax import lax
from jax.experimental import pallas as pl
from jax.experimental.pallas import tpu as pltpu

Fh = 64
Fv = 64
Fe = 16
B_EVENTS = 4096
N_SLOTS = 512
BB = 8
NB = B_EVENTS // BB


def _gru_gates(gi, gh, h):
    i_r, i_z, i_n = gi[:, :Fh], gi[:, Fh:2 * Fh], gi[:, 2 * Fh:]
    h_r, h_z, h_n = gh[:, :Fh], gh[:, Fh:2 * Fh], gh[:, 2 * Fh:]
    r = jax.nn.sigmoid(i_r + h_r)
    z = jax.nn.sigmoid(i_z + h_z)
    n = jnp.tanh(i_n + r * h_n)
    return (1.0 - z) * n + z * h


def _fused_body(es_ref, ed_ref, ht_ref, xsh_ref,
                w1o_ref, w2o_ref, who_ref, bio_ref, bho_ref,
                w1i_ref, w2i_ref, whi_ref, bii_ref, bhi_ref,
                hsn_ref, hdn_ref, out_ref):
    lane = lax.broadcasted_iota(jnp.int32, (1, N_SLOTS), 1)
    hs_rows = []
    hd_rows = []
    for i in range(BB):
        s = es_ref[0, 0, i]
        t = ed_ref[0, 0, i]
        blk = ht_ref[i]
        hs_rows.append(jnp.sum(jnp.where(lane == s, blk, 0.0), axis=1))
        hd_rows.append(jnp.sum(jnp.where(lane == t, blk, 0.0), axis=1))
    hs = jnp.stack(hs_rows, axis=0)
    hd = jnp.stack(hd_rows, axis=0)
    xsh = xsh_ref[...]
    f32 = jnp.float32
    gi = (jnp.dot(hd, w1o_ref[...], preferred_element_type=f32)
          + jnp.dot(xsh, w2o_ref[...], preferred_element_type=f32)
          + bio_ref[...])
    gh = jnp.dot(hs, who_ref[...], preferred_element_type=f32) + bho_ref[...]
    hs_new = _gru_gates(gi, gh, hs)
    hsn_ref[...] = hs_new
    gi = (jnp.dot(hs, w1i_ref[...], preferred_element_type=f32)
          + jnp.dot(xsh, w2i_ref[...], preferred_element_type=f32)
          + bii_ref[...])
    gh = jnp.dot(hd, whi_ref[...], preferred_element_type=f32) + bhi_ref[...]
    hd_new = _gru_gates(gi, gh, hd)
    hdn_ref[...] = hd_new
    hs_cols = hs_new.T
    hd_cols = hd_new.T
    for i in range(BB):
        s = es_ref[0, 0, i]
        t = ed_ref[0, 0, i]
        blk = ht_ref[i]
        blk = jnp.where(lane == s, hs_cols[:, i:i + 1], blk)
        blk = jnp.where(lane == t, hd_cols[:, i:i + 1], blk)
        out_ref[i] = blk


def kernel(E, Xe, Xv, H, Wih_out, Whh_out, bih_out, bhh_out,
           Wih_in, Whh_in, bih_in, bhh_in):
    esrc = E[:, 0]
    edst = E[:, 1]
    ht = jnp.swapaxes(H, 1, 2)
    xshared = jnp.concatenate([Xv[:, 0, :], Xv[:, 1, :], Xe], axis=1)
    w1o = Wih_out[:, :Fh].T
    w2o = Wih_out[:, Fh:].T
    w1i = Wih_in[:, :Fh].T
    w2i = Wih_in[:, Fh:].T
    e3_src = esrc.reshape(NB, 1, BB)
    e3_dst = edst.reshape(NB, 1, BB)

    smem_blk = pl.BlockSpec((1, 1, BB), lambda b: (b, 0, 0),
                            memory_space=pltpu.SMEM)
    full = lambda a: pl.BlockSpec(a.shape, lambda b: (0,) * a.ndim)
    f32 = jnp.float32
    args = (w1o, w2o, Whh_out.T, bih_out[None, :], bhh_out[None, :],
            w1i, w2i, Whh_in.T, bih_in[None, :], bhh_in[None, :])
    hsrc_new, hdst_new, out_t = pl.pallas_call(
        _fused_body,
        grid=(NB,),
        in_specs=[smem_blk, smem_blk,
                  pl.BlockSpec((BB, Fh, N_SLOTS), lambda b: (b, 0, 0)),
                  pl.BlockSpec((BB, 2 * Fv + Fe), lambda b: (b, 0))]
                 + [full(a) for a in args],
        out_specs=[pl.BlockSpec((BB, Fh), lambda b: (b, 0)),
                   pl.BlockSpec((BB, Fh), lambda b: (b, 0)),
                   pl.BlockSpec((BB, Fh, N_SLOTS), lambda b: (b, 0, 0))],
        out_shape=[jax.ShapeDtypeStruct((B_EVENTS, Fh), f32),
                   jax.ShapeDtypeStruct((B_EVENTS, Fh), f32),
                   jax.ShapeDtypeStruct((B_EVENTS, Fh, N_SLOTS), f32)],
        compiler_params=pltpu.CompilerParams(
            dimension_semantics=("arbitrary",)),
    )(e3_src, e3_dst, ht, xshared, *args)
    return (hsrc_new, hdst_new, jnp.swapaxes(out_t, 1, 2))


# speedup vs baseline: 6.7711x; 1.4137x over previous
import jax
import jax.numpy as jnp
from jax import lax
from jax.experimental import pallas as pl
from jax.experimental.pallas import tpu as pltpu

Fh = 64
Fv = 64
Fe = 16
B_EVENTS = 4096
N_SLOTS = 512
BB = 16
NB = B_EVENTS // BB


def _gru_gates(gi, gh, h):
    i_r, i_z, i_n = gi[:, :Fh], gi[:, Fh:2 * Fh], gi[:, 2 * Fh:]
    h_r, h_z, h_n = gh[:, :Fh], gh[:, Fh:2 * Fh], gh[:, 2 * Fh:]
    r = jax.nn.sigmoid(i_r + h_r)
    z = jax.nn.sigmoid(i_z + h_z)
    n = jnp.tanh(i_n + r * h_n)
    return (1.0 - z) * n + z * h


def _fused_body(es_ref, ed_ref, ht_ref, xsh_ref,
                w1o_ref, w2o_ref, who_ref, bio_ref, bho_ref,
                w1i_ref, w2i_ref, whi_ref, bii_ref, bhi_ref,
                hsn_ref, hdn_ref, out_ref):
    lane = lax.broadcasted_iota(jnp.int32, (1, N_SLOTS), 1)
    hs_rows = []
    hd_rows = []
    for i in range(BB):
        s = es_ref[0, 0, i]
        t = ed_ref[0, 0, i]
        blk = ht_ref[i]
        hs_rows.append(jnp.sum(jnp.where(lane == s, blk, 0.0), axis=1))
        hd_rows.append(jnp.sum(jnp.where(lane == t, blk, 0.0), axis=1))
    hs = jnp.stack(hs_rows, axis=0)
    hd = jnp.stack(hd_rows, axis=0)
    xsh = xsh_ref[...]
    f32 = jnp.float32
    gi = (jnp.dot(hd, w1o_ref[...], preferred_element_type=f32)
          + jnp.dot(xsh, w2o_ref[...], preferred_element_type=f32)
          + bio_ref[...])
    gh = jnp.dot(hs, who_ref[...], preferred_element_type=f32) + bho_ref[...]
    hs_new = _gru_gates(gi, gh, hs)
    hsn_ref[...] = hs_new
    gi = (jnp.dot(hs, w1i_ref[...], preferred_element_type=f32)
          + jnp.dot(xsh, w2i_ref[...], preferred_element_type=f32)
          + bii_ref[...])
    gh = jnp.dot(hd, whi_ref[...], preferred_element_type=f32) + bhi_ref[...]
    hd_new = _gru_gates(gi, gh, hd)
    hdn_ref[...] = hd_new
    hs_cols = hs_new.T
    hd_cols = hd_new.T
    for i in range(BB):
        s = es_ref[0, 0, i]
        t = ed_ref[0, 0, i]
        blk = ht_ref[i]
        blk = jnp.where(lane == s, hs_cols[:, i:i + 1], blk)
        blk = jnp.where(lane == t, hd_cols[:, i:i + 1], blk)
        out_ref[i] = blk


def kernel(E, Xe, Xv, H, Wih_out, Whh_out, bih_out, bhh_out,
           Wih_in, Whh_in, bih_in, bhh_in):
    esrc = E[:, 0]
    edst = E[:, 1]
    ht = jnp.swapaxes(H, 1, 2)
    xshared = jnp.concatenate([Xv[:, 0, :], Xv[:, 1, :], Xe], axis=1)
    w1o = Wih_out[:, :Fh].T
    w2o = Wih_out[:, Fh:].T
    w1i = Wih_in[:, :Fh].T
    w2i = Wih_in[:, Fh:].T
    e3_src = esrc.reshape(NB, 1, BB)
    e3_dst = edst.reshape(NB, 1, BB)

    smem_blk = pl.BlockSpec((1, 1, BB), lambda b: (b, 0, 0),
                            memory_space=pltpu.SMEM)
    full = lambda a: pl.BlockSpec(a.shape, lambda b: (0,) * a.ndim)
    f32 = jnp.float32
    args = (w1o, w2o, Whh_out.T, bih_out[None, :], bhh_out[None, :],
            w1i, w2i, Whh_in.T, bih_in[None, :], bhh_in[None, :])
    hsrc_new, hdst_new, out_t = pl.pallas_call(
        _fused_body,
        grid=(NB,),
        in_specs=[smem_blk, smem_blk,
                  pl.BlockSpec((BB, Fh, N_SLOTS), lambda b: (b, 0, 0)),
                  pl.BlockSpec((BB, 2 * Fv + Fe), lambda b: (b, 0))]
                 + [full(a) for a in args],
        out_specs=[pl.BlockSpec((BB, Fh), lambda b: (b, 0)),
                   pl.BlockSpec((BB, Fh), lambda b: (b, 0)),
                   pl.BlockSpec((BB, Fh, N_SLOTS), lambda b: (b, 0, 0))],
        out_shape=[jax.ShapeDtypeStruct((B_EVENTS, Fh), f32),
                   jax.ShapeDtypeStruct((B_EVENTS, Fh), f32),
                   jax.ShapeDtypeStruct((B_EVENTS, Fh, N_SLOTS), f32)],
        compiler_params=pltpu.CompilerParams(
            dimension_semantics=("arbitrary",)),
    )(e3_src, e3_dst, ht, xshared, *args)
    return (hsrc_new, hdst_new, jnp.swapaxes(out_t, 1, 2))


# speedup vs baseline: 8.3363x; 1.2312x over previous
import jax
import jax.numpy as jnp
from jax import lax
from jax.experimental import pallas as pl
from jax.experimental.pallas import tpu as pltpu

Fh = 64
Fv = 64
Fe = 16
B_EVENTS = 4096
N_SLOTS = 512
BB = 32
NB = B_EVENTS // BB


def _gru_gates(gi, gh, h):
    i_r, i_z, i_n = gi[:, :Fh], gi[:, Fh:2 * Fh], gi[:, 2 * Fh:]
    h_r, h_z, h_n = gh[:, :Fh], gh[:, Fh:2 * Fh], gh[:, 2 * Fh:]
    r = jax.nn.sigmoid(i_r + h_r)
    z = jax.nn.sigmoid(i_z + h_z)
    n = jnp.tanh(i_n + r * h_n)
    return (1.0 - z) * n + z * h


def _fused_body(es_ref, ed_ref, ht_ref, xsh_ref,
                w1o_ref, w2o_ref, who_ref, bio_ref, bho_ref,
                w1i_ref, w2i_ref, whi_ref, bii_ref, bhi_ref,
                hsn_ref, hdn_ref, out_ref):
    lane = lax.broadcasted_iota(jnp.int32, (1, N_SLOTS), 1)
    hs_rows = []
    hd_rows = []
    for i in range(BB):
        s = es_ref[0, 0, i]
        t = ed_ref[0, 0, i]
        blk = ht_ref[i]
        hs_rows.append(jnp.sum(jnp.where(lane == s, blk, 0.0), axis=1))
        hd_rows.append(jnp.sum(jnp.where(lane == t, blk, 0.0), axis=1))
    hs = jnp.stack(hs_rows, axis=0)
    hd = jnp.stack(hd_rows, axis=0)
    xsh = xsh_ref[...]
    f32 = jnp.float32
    gi = (jnp.dot(hd, w1o_ref[...], preferred_element_type=f32)
          + jnp.dot(xsh, w2o_ref[...], preferred_element_type=f32)
          + bio_ref[...])
    gh = jnp.dot(hs, who_ref[...], preferred_element_type=f32) + bho_ref[...]
    hs_new = _gru_gates(gi, gh, hs)
    hsn_ref[...] = hs_new
    gi = (jnp.dot(hs, w1i_ref[...], preferred_element_type=f32)
          + jnp.dot(xsh, w2i_ref[...], preferred_element_type=f32)
          + bii_ref[...])
    gh = jnp.dot(hd, whi_ref[...], preferred_element_type=f32) + bhi_ref[...]
    hd_new = _gru_gates(gi, gh, hd)
    hdn_ref[...] = hd_new
    hs_cols = hs_new.T
    hd_cols = hd_new.T
    for i in range(BB):
        s = es_ref[0, 0, i]
        t = ed_ref[0, 0, i]
        blk = ht_ref[i]
        blk = jnp.where(lane == s, hs_cols[:, i:i + 1], blk)
        blk = jnp.where(lane == t, hd_cols[:, i:i + 1], blk)
        out_ref[i] = blk


def kernel(E, Xe, Xv, H, Wih_out, Whh_out, bih_out, bhh_out,
           Wih_in, Whh_in, bih_in, bhh_in):
    esrc = E[:, 0]
    edst = E[:, 1]
    ht = jnp.swapaxes(H, 1, 2)
    xshared = jnp.concatenate([Xv[:, 0, :], Xv[:, 1, :], Xe], axis=1)
    w1o = Wih_out[:, :Fh].T
    w2o = Wih_out[:, Fh:].T
    w1i = Wih_in[:, :Fh].T
    w2i = Wih_in[:, Fh:].T
    e3_src = esrc.reshape(NB, 1, BB)
    e3_dst = edst.reshape(NB, 1, BB)

    smem_blk = pl.BlockSpec((1, 1, BB), lambda b: (b, 0, 0),
                            memory_space=pltpu.SMEM)
    full = lambda a: pl.BlockSpec(a.shape, lambda b: (0,) * a.ndim)
    f32 = jnp.float32
    args = (w1o, w2o, Whh_out.T, bih_out[None, :], bhh_out[None, :],
            w1i, w2i, Whh_in.T, bih_in[None, :], bhh_in[None, :])
    hsrc_new, hdst_new, out_t = pl.pallas_call(
        _fused_body,
        grid=(NB,),
        in_specs=[smem_blk, smem_blk,
                  pl.BlockSpec((BB, Fh, N_SLOTS), lambda b: (b, 0, 0)),
                  pl.BlockSpec((BB, 2 * Fv + Fe), lambda b: (b, 0))]
                 + [full(a) for a in args],
        out_specs=[pl.BlockSpec((BB, Fh), lambda b: (b, 0)),
                   pl.BlockSpec((BB, Fh), lambda b: (b, 0)),
                   pl.BlockSpec((BB, Fh, N_SLOTS), lambda b: (b, 0, 0))],
        out_shape=[jax.ShapeDtypeStruct((B_EVENTS, Fh), f32),
                   jax.ShapeDtypeStruct((B_EVENTS, Fh), f32),
                   jax.ShapeDtypeStruct((B_EVENTS, Fh, N_SLOTS), f32)],
        compiler_params=pltpu.CompilerParams(
            dimension_semantics=("arbitrary",)),
    )(e3_src, e3_dst, ht, xshared, *args)
    return (hsrc_new, hdst_new, jnp.swapaxes(out_t, 1, 2))


# speedup vs baseline: 9.1601x; 1.0988x over previous
import jax
import jax.numpy as jnp
from jax import lax
from jax.experimental import pallas as pl
from jax.experimental.pallas import tpu as pltpu

Fh = 64
Fv = 64
Fe = 16
B_EVENTS = 4096
N_SLOTS = 512
BB = 64
NB = B_EVENTS // BB


def _gru_gates(gi, gh, h):
    i_r, i_z, i_n = gi[:, :Fh], gi[:, Fh:2 * Fh], gi[:, 2 * Fh:]
    h_r, h_z, h_n = gh[:, :Fh], gh[:, Fh:2 * Fh], gh[:, 2 * Fh:]
    r = jax.nn.sigmoid(i_r + h_r)
    z = jax.nn.sigmoid(i_z + h_z)
    n = jnp.tanh(i_n + r * h_n)
    return (1.0 - z) * n + z * h


def _fused_body(es_ref, ed_ref, ht_ref, xsh_ref,
                w1o_ref, w2o_ref, who_ref, bio_ref, bho_ref,
                w1i_ref, w2i_ref, whi_ref, bii_ref, bhi_ref,
                hsn_ref, hdn_ref, out_ref):
    lane = lax.broadcasted_iota(jnp.int32, (1, N_SLOTS), 1)
    hs_rows = []
    hd_rows = []
    for i in range(BB):
        s = es_ref[0, 0, i]
        t = ed_ref[0, 0, i]
        blk = ht_ref[i]
        hs_rows.append(jnp.sum(jnp.where(lane == s, blk, 0.0), axis=1))
        hd_rows.append(jnp.sum(jnp.where(lane == t, blk, 0.0), axis=1))
    hs = jnp.stack(hs_rows, axis=0)
    hd = jnp.stack(hd_rows, axis=0)
    xsh = xsh_ref[...]
    f32 = jnp.float32
    gi = (jnp.dot(hd, w1o_ref[...], preferred_element_type=f32)
          + jnp.dot(xsh, w2o_ref[...], preferred_element_type=f32)
          + bio_ref[...])
    gh = jnp.dot(hs, who_ref[...], preferred_element_type=f32) + bho_ref[...]
    hs_new = _gru_gates(gi, gh, hs)
    hsn_ref[...] = hs_new
    gi = (jnp.dot(hs, w1i_ref[...], preferred_element_type=f32)
          + jnp.dot(xsh, w2i_ref[...], preferred_element_type=f32)
          + bii_ref[...])
    gh = jnp.dot(hd, whi_ref[...], preferred_element_type=f32) + bhi_ref[...]
    hd_new = _gru_gates(gi, gh, hd)
    hdn_ref[...] = hd_new
    hs_cols = hs_new.T
    hd_cols = hd_new.T
    for i in range(BB):
        s = es_ref[0, 0, i]
        t = ed_ref[0, 0, i]
        blk = ht_ref[i]
        blk = jnp.where(lane == s, hs_cols[:, i:i + 1], blk)
        blk = jnp.where(lane == t, hd_cols[:, i:i + 1], blk)
        out_ref[i] = blk


def kernel(E, Xe, Xv, H, Wih_out, Whh_out, bih_out, bhh_out,
           Wih_in, Whh_in, bih_in, bhh_in):
    esrc = E[:, 0]
    edst = E[:, 1]
    ht = jnp.swapaxes(H, 1, 2)
    xshared = jnp.concatenate([Xv[:, 0, :], Xv[:, 1, :], Xe], axis=1)
    w1o = Wih_out[:, :Fh].T
    w2o = Wih_out[:, Fh:].T
    w1i = Wih_in[:, :Fh].T
    w2i = Wih_in[:, Fh:].T
    e3_src = esrc.reshape(NB, 1, BB)
    e3_dst = edst.reshape(NB, 1, BB)

    smem_blk = pl.BlockSpec((1, 1, BB), lambda b: (b, 0, 0),
                            memory_space=pltpu.SMEM)
    full = lambda a: pl.BlockSpec(a.shape, lambda b: (0,) * a.ndim)
    f32 = jnp.float32
    args = (w1o, w2o, Whh_out.T, bih_out[None, :], bhh_out[None, :],
            w1i, w2i, Whh_in.T, bih_in[None, :], bhh_in[None, :])
    hsrc_new, hdst_new, out_t = pl.pallas_call(
        _fused_body,
        grid=(NB,),
        in_specs=[smem_blk, smem_blk,
                  pl.BlockSpec((BB, Fh, N_SLOTS), lambda b: (b, 0, 0)),
                  pl.BlockSpec((BB, 2 * Fv + Fe), lambda b: (b, 0))]
                 + [full(a) for a in args],
        out_specs=[pl.BlockSpec((BB, Fh), lambda b: (b, 0)),
                   pl.BlockSpec((BB, Fh), lambda b: (b, 0)),
                   pl.BlockSpec((BB, Fh, N_SLOTS), lambda b: (b, 0, 0))],
        out_shape=[jax.ShapeDtypeStruct((B_EVENTS, Fh), f32),
                   jax.ShapeDtypeStruct((B_EVENTS, Fh), f32),
                   jax.ShapeDtypeStruct((B_EVENTS, Fh, N_SLOTS), f32)],
        compiler_params=pltpu.CompilerParams(
            dimension_semantics=("arbitrary",)),
    )(e3_src, e3_dst, ht, xshared, *args)
    return (hsrc_new, hdst_new, jnp.swapaxes(out_t, 1, 2))
